# Initial kernel scaffold; baseline (speedup 1.0000x reference)
import jax
import jax.numpy as jnp
from jax.experimental import pallas as pl


def kernel(q_xyz, sup_xyz, q_mask, sup_mask, sup_feats, W_fi1, W_fi2, W_fi3, W_df1, W_df2, W_df3):
    raise NotImplementedError("write your pallas kernel here")


# speedup vs baseline: 4.0388x; 4.0388x over previous
import functools

import jax
import jax.numpy as jnp
from jax import lax
from jax.experimental import pallas as pl
from jax.experimental.pallas import tpu as pltpu
from jax.experimental.pallas import tpu_sc as plsc

RADIUS = 0.2
R2 = RADIUS * RADIUS
M = 32
NGEO = 19
NGEO_PAD = 24
EPS_BN = 1e-5

BN_A = 256
BN_B = 128
CHUNK = 128

SC_CORES = 2
SC_SUBCORES = 16
SC_WORKERS = SC_CORES * SC_SUBCORES
SC_CHUNK = 512


def _pass_a_body(q_ref, supT_ref, A_ref, feats_ref, idx_ref, stats_ref):
    b = pl.program_id(0)
    BN = q_ref.shape[1]
    NS = supT_ref.shape[2]

    q = q_ref[0]
    qx = q[:, 0:1]
    qy = q[:, 1:2]
    qz = q[:, 2:3]
    sx = supT_ref[0, 0:1, :]
    sy = supT_ref[0, 1:2, :]
    sz = supT_ref[0, 2:3, :]

    dx = qx - sx
    dy = qy - sy
    dz = qz - sz
    d2 = (dx * dx + dy * dy) + dz * dz
    valid = (d2 <= R2).astype(jnp.float32)

    nchunks = NS // CHUNK
    row_i = lax.broadcasted_iota(jnp.int32, (CHUNK, CHUNK), 0)
    col_i = lax.broadcasted_iota(jnp.int32, (CHUNK, CHUNK), 1)
    lt = (row_i <= col_i).astype(jnp.float32)
    off = jnp.zeros((BN, 1), jnp.float32)
    rank_chunks = []
    for c in range(nchunks):
        vc = valid[:, c * CHUNK:(c + 1) * CHUNK]
        rc = jnp.dot(vc, lt, preferred_element_type=jnp.float32, precision=lax.Precision.HIGHEST) + off
        rank_chunks.append(rc)
        off = rc[:, CHUNK - 1:CHUNK]
    rank = jnp.concatenate(rank_chunks, axis=1)
    cnt = off

    rankv = jnp.where(valid > 0.5, rank, -1.0)

    A = A_ref[0]
    gx_l, gy_l, gz_l, li_l = [], [], [], []
    for m in range(M):
        sm = (jnp.abs(rankv - (m + 1.0)) < 0.5).astype(jnp.float32)
        gm = jnp.dot(sm, A, preferred_element_type=jnp.float32, precision=lax.Precision.HIGHEST)
        gx_l.append(gm[:, 0:1])
        gy_l.append(gm[:, 1:2])
        gz_l.append(gm[:, 2:3])
        li_l.append(gm[:, 3:4])
    gx = jnp.concatenate(gx_l, axis=1)
    gy = jnp.concatenate(gy_l, axis=1)
    gz = jnp.concatenate(gz_l, axis=1)
    li = jnp.concatenate(li_l, axis=1)

    slot = lax.broadcasted_iota(jnp.int32, (BN, M), 1).astype(jnp.float32)
    mask = (slot < cnt - 0.5).astype(jnp.float32)

    li0 = li[:, 0:1]
    li_fix = jnp.where(mask > 0.5, li, li0)
    flat = li_fix + b.astype(jnp.float32) * float(NS) + 0.5
    idx_ref[0] = flat.astype(jnp.int32)

    rx = gx - qx
    ry = gy - qy
    rz = gz - qz
    d2m = (rx * rx + ry * ry) + rz * rz
    dist = jnp.sqrt(d2m + 1e-12)
    invd = 1.0 / (dist + 1e-8)
    cntm = jnp.maximum(jnp.sum(mask, axis=1, keepdims=True), 1.0)
    cx = jnp.sum(rx * mask, axis=1, keepdims=True) / cntm
    cy = jnp.sum(ry * mask, axis=1, keepdims=True) / cntm
    cz = jnp.sum(rz * mask, axis=1, keepdims=True) / cntm
    ones = jnp.ones_like(rx)
    feats = [
        rx / RADIUS,
        ry / RADIUS,
        rz / RADIUS,
        dist / RADIUS,
        rx * invd,
        ry * invd,
        rz * invd,
        ry * cz - rz * cy,
        rz * cx - rx * cz,
        rx * cy - ry * cx,
        (rx * cx + ry * cy) + rz * cz,
        rx * rx / R2,
        ry * ry / R2,
        rz * rz / R2,
        cx * ones,
        cy * ones,
        cz * ones,
        jnp.sqrt((cx * cx + cy * cy) + cz * cz + 1e-12) * ones,
        dist * dist / R2,
    ]
    zrow = jnp.zeros((BN, 1, M), jnp.float32)
    planes = []
    srows = []
    qrows = []
    for f in feats:
        fm = f * mask
        planes.append(fm[:, None, :])
        srows.append(jnp.sum(fm, axis=0, keepdims=True))
        qrows.append(jnp.sum(fm * fm, axis=0, keepdims=True))
    for _ in range(NGEO_PAD - NGEO):
        planes.append(zrow)
        srows.append(jnp.zeros((1, M), jnp.float32))
        qrows.append(jnp.zeros((1, M), jnp.float32))
    feats_ref[0] = jnp.concatenate(planes, axis=1)

    stats_upd = jnp.concatenate(srows + qrows, axis=0)
    first = (pl.program_id(0) == 0) & (pl.program_id(1) == 0)

    @pl.when(first)
    def _():
        stats_ref[...] = jnp.zeros_like(stats_ref)

    stats_ref[...] += stats_upd


def _pass_b_body(g_ref, f_ref, wr_ref, wf_ref, wfi_ref, bias_ref,
                 y_ref, stats_ref):
    BN = g_ref.shape[1]
    CP = g_ref.shape[3]
    g = g_ref[0]
    ce = g[:, 0:1, :]
    r = (g - ce).reshape(BN * M, CP)
    f = f_ref[0].reshape(BN * M, NGEO_PAD)
    y = (jnp.dot(r, wr_ref[...], preferred_element_type=jnp.float32, precision=lax.Precision.HIGHEST)
         + jnp.dot(f, wf_ref[...], preferred_element_type=jnp.float32, precision=lax.Precision.HIGHEST))
    ceo = jnp.dot(g[:, 0, :], wfi_ref[...],
                  preferred_element_type=jnp.float32, precision=lax.Precision.HIGHEST)
    y3 = (y.reshape(BN, M, 64) + ceo[:, None, :]
          + bias_ref[0:1, :][None, :, :])

    ymax = y3[:, 0, :]
    for m in range(1, M):
        ymax = jnp.maximum(ymax, y3[:, m, :])
    y_ref[0] = ymax

    y2 = y3.reshape(BN * M, 64)
    s = jnp.sum(y2, axis=0, keepdims=True)
    ss = jnp.sum(y2 * y2, axis=0, keepdims=True)
    upd = jnp.concatenate(
        [s, ss, jnp.zeros((6, 64), jnp.float32)], axis=0)
    first = (pl.program_id(0) == 0) & (pl.program_id(1) == 0)

    @pl.when(first)
    def _():
        stats_ref[...] = jnp.zeros_like(stats_ref)

    stats_ref[...] += upd


def _pass_c_body(y_ref, scsh_ref, o_ref):
    sc = scsh_ref[0:1, :]
    sh = scsh_ref[1:2, :]
    o_ref[0] = jnp.maximum(y_ref[0] * sc + sh, 0.0)


def _sc_gather(table, flat_idx):
    TOT = flat_idx.shape[0]
    D = table.shape[1]
    per_w = TOT // SC_WORKERS
    nchunks = per_w // SC_CHUNK
    mesh = plsc.VectorSubcoreMesh(core_axis_name="c", subcore_axis_name="s")

    @functools.partial(
        pl.kernel,
        out_type=jax.ShapeDtypeStruct((TOT, D), jnp.float32),
        mesh=mesh,
        scratch_types=[
            pltpu.VMEM((SC_CHUNK,), jnp.int32),
            pltpu.VMEM((SC_CHUNK, D), jnp.float32),
            pltpu.SemaphoreType.DMA,
        ],
    )
    def gather_k(table_hbm, idx_hbm, out_hbm, idx_v, rows_v, sem):
        wid = lax.axis_index("s") * SC_CORES + lax.axis_index("c")
        base = wid * per_w

        def body(k, carry):
            start = base + k * SC_CHUNK
            pltpu.sync_copy(idx_hbm.at[pl.ds(start, SC_CHUNK)], idx_v)
            pltpu.async_copy(table_hbm.at[idx_v], rows_v, sem).wait()
            pltpu.sync_copy(rows_v, out_hbm.at[pl.ds(start, SC_CHUNK)])
            return carry

        lax.fori_loop(0, nchunks, body, 0)

    return gather_k(table, flat_idx)


def kernel(q_xyz, sup_xyz, q_mask, sup_mask, sup_feats,
           W_fi1, W_fi2, W_fi3, W_df1, W_df2, W_df3):
    B, N, _ = q_xyz.shape
    NS = sup_xyz.shape[1]
    C = sup_feats.shape[1]
    f32 = jnp.float32

    supT = jnp.transpose(sup_xyz, (0, 2, 1))
    iota = jnp.arange(NS, dtype=f32)[None, :, None]
    A = jnp.concatenate(
        [sup_xyz, jnp.broadcast_to(iota, (B, NS, 1)),
         jnp.zeros((B, NS, 4), f32)], axis=-1)

    nb_a = N // BN_A
    feats_cm, idx, stats1 = pl.pallas_call(
        _pass_a_body,
        grid=(B, nb_a),
        in_specs=[
            pl.BlockSpec((1, BN_A, 3), lambda b, n: (b, n, 0)),
            pl.BlockSpec((1, 3, NS), lambda b, n: (b, 0, 0)),
            pl.BlockSpec((1, NS, 8), lambda b, n: (b, 0, 0)),
        ],
        out_specs=[
            pl.BlockSpec((1, BN_A, NGEO_PAD, M), lambda b, n: (b, n, 0, 0)),
            pl.BlockSpec((1, BN_A, M), lambda b, n: (b, n, 0)),
            pl.BlockSpec((48, M), lambda b, n: (0, 0)),
        ],
        out_shape=[
            jax.ShapeDtypeStruct((B, N, NGEO_PAD, M), f32),
            jax.ShapeDtypeStruct((B, N, M), jnp.int32),
            jax.ShapeDtypeStruct((48, M), f32),
        ],
    )(q_xyz, supT, A)

    cnt_tot = float(B * N * M)
    s1 = jnp.sum(stats1[0:NGEO_PAD], axis=1)[:NGEO]
    q1 = jnp.sum(stats1[NGEO_PAD:2 * NGEO_PAD], axis=1)[:NGEO]
    mu1 = s1 / cnt_tot
    var1 = q1 / cnt_tot - mu1 * mu1
    inv1 = 1.0 / jnp.sqrt(var1 + EPS_BN)

    c3o = W_fi1.shape[0]
    co3 = W_fi3.shape[0]
    c3i = W_fi1.shape[1]
    ci3 = W_fi3.shape[1]
    Cout = 2 * c3o + co3

    WrT = jnp.zeros((128, Cout), f32)
    WrT = WrT.at[0:c3i, 0:c3o].set(W_df1[:, 0:c3i].T)
    WrT = WrT.at[c3i:2 * c3i, c3o:2 * c3o].set(W_df2[:, 0:c3i].T)
    WrT = WrT.at[2 * c3i:C, 2 * c3o:Cout].set(W_df3[:, 0:ci3].T)

    WfT = jnp.zeros((NGEO_PAD, Cout), f32)
    WfT = WfT.at[0:3, 0:c3o].set((W_df1[:, c3i:c3i + 3] * inv1[None, 0:3]).T)
    WfT = WfT.at[3:7, c3o:2 * c3o].set(
        (W_df2[:, c3i:c3i + 4] * inv1[None, 3:7]).T)
    WfT = WfT.at[7:NGEO, 2 * c3o:Cout].set(
        (W_df3[:, ci3:ci3 + 12] * inv1[None, 7:NGEO]).T)
    mu_pad = jnp.zeros((NGEO_PAD,), f32).at[0:NGEO].set(mu1)
    bias = -(mu_pad @ WfT)
    bias8 = jnp.zeros((8, Cout), f32).at[0, :].set(bias)

    WfiT = jnp.zeros((128, Cout), f32)
    WfiT = WfiT.at[0:c3i, 0:c3o].set(W_fi1.T)
    WfiT = WfiT.at[c3i:2 * c3i, c3o:2 * c3o].set(W_fi2.T)
    WfiT = WfiT.at[2 * c3i:C, 2 * c3o:Cout].set(W_fi3.T)

    table = jnp.zeros((B * NS, 128), f32).at[:, 0:C].set(
        jnp.transpose(sup_feats, (0, 2, 1)).reshape(B * NS, C))
    g = _sc_gather(table, idx.reshape(B * N * M))
    gfeat = g.reshape(B, N, M, 128)
    feats24 = jnp.transpose(feats_cm, (0, 1, 3, 2))

    nb_b = N // BN_B
    y, stats2 = pl.pallas_call(
        _pass_b_body,
        grid=(B, nb_b),
        in_specs=[
            pl.BlockSpec((1, BN_B, M, 128), lambda b, n: (b, n, 0, 0)),

            pl.BlockSpec((1, BN_B, M, NGEO_PAD), lambda b, n: (b, n, 0, 0)),
            pl.BlockSpec((128, Cout), lambda b, n: (0, 0)),
            pl.BlockSpec((NGEO_PAD, Cout), lambda b, n: (0, 0)),
            pl.BlockSpec((128, Cout), lambda b, n: (0, 0)),
            pl.BlockSpec((8, Cout), lambda b, n: (0, 0)),
        ],
        out_specs=[
            pl.BlockSpec((1, BN_B, Cout), lambda b, n: (b, n, 0)),
            pl.BlockSpec((8, Cout), lambda b, n: (0, 0)),
        ],
        out_shape=[
            jax.ShapeDtypeStruct((B, N, Cout), f32),
            jax.ShapeDtypeStruct((8, Cout), f32),
        ],
    )(gfeat, feats24, WrT, WfT, WfiT, bias8)

    mu2 = stats2[0] / cnt_tot
    var2 = stats2[1] / cnt_tot - mu2 * mu2
    sc2 = 1.0 / jnp.sqrt(var2 + EPS_BN)
    scsh = jnp.zeros((8, Cout), f32).at[0, :].set(sc2).at[1, :].set(-mu2 * sc2)

    z = pl.pallas_call(
        _pass_c_body,
        grid=(B,),
        in_specs=[
            pl.BlockSpec((1, N, Cout), lambda b: (b, 0, 0)),
            pl.BlockSpec((8, Cout), lambda b: (0, 0)),
        ],
        out_specs=pl.BlockSpec((1, N, Cout), lambda b: (b, 0, 0)),
        out_shape=jax.ShapeDtypeStruct((B, N, Cout), f32),
    )(y, scsh)

    return jnp.transpose(z, (0, 2, 1))


# speedup vs baseline: 8.2958x; 2.0540x over previous
import functools

import jax
import jax.numpy as jnp
from jax import lax
from jax.experimental import pallas as pl
from jax.experimental.pallas import tpu as pltpu
from jax.experimental.pallas import tpu_sc as plsc

RADIUS = 0.2
R2 = RADIUS * RADIUS
M = 32
NGEO = 19
NGEO_PAD = 24
EPS_BN = 1e-5

BN_A = 256
BN_B = 128
CHUNK = 128

SC_CORES = 2
SC_SUBCORES = 16
SC_WORKERS = SC_CORES * SC_SUBCORES
SC_CHUNK = 512


def _pass_a_body(q_ref, supT_ref, idx_ref, cnt_ref):
    b = pl.program_id(0)
    BN = q_ref.shape[1]
    NS = supT_ref.shape[2]

    q = q_ref[0]
    qx = q[:, 0:1]
    qy = q[:, 1:2]
    qz = q[:, 2:3]
    sx = supT_ref[0, 0:1, :]
    sy = supT_ref[0, 1:2, :]
    sz = supT_ref[0, 2:3, :]

    dx = qx - sx
    dy = qy - sy
    dz = qz - sz
    d2 = (dx * dx + dy * dy) + dz * dz
    valid = (d2 <= R2).astype(jnp.bfloat16)

    nchunks = NS // CHUNK
    row_i = lax.broadcasted_iota(jnp.int32, (CHUNK, CHUNK), 0)
    col_i = lax.broadcasted_iota(jnp.int32, (CHUNK, CHUNK), 1)
    lt = (row_i <= col_i).astype(jnp.bfloat16)
    off = jnp.zeros((BN, 1), jnp.float32)
    rank_chunks = []
    for c in range(nchunks):
        vc = valid[:, c * CHUNK:(c + 1) * CHUNK]
        rc = jnp.dot(vc, lt, preferred_element_type=jnp.float32) + off
        rank_chunks.append(rc)
        off = rc[:, CHUNK - 1:CHUNK]
    rank = jnp.concatenate(rank_chunks, axis=1)
    cnt = off

    rcl = jnp.minimum(rank, 33.0)
    idx_cols = []
    for m in range(M):
        cmp = (rcl <= (m + 0.5)).astype(jnp.float32)
        idx_cols.append(jnp.sum(cmp, axis=1, keepdims=True))
    idxm = jnp.concatenate(idx_cols, axis=1)

    slot = lax.broadcasted_iota(jnp.int32, (BN, M), 1).astype(jnp.float32)
    mask = (slot < cnt - 0.5).astype(jnp.float32)

    idx0 = jnp.where(cnt > 0.5, idxm[:, 0:1], 0.0)
    idx_fix = jnp.where(mask > 0.5, idxm, idx0)
    flat = idx_fix + b.astype(jnp.float32) * float(NS) + 0.5
    idx_ref[0] = flat.astype(jnp.int32)
    cnt_ref[0] = cnt


def _pass_b1_body(q_ref, g_ref, cnt_ref, feats_ref, stats_ref):
    BN = q_ref.shape[1]
    q = q_ref[0]
    qx = q[:, 0:1]
    qy = q[:, 1:2]
    qz = q[:, 2:3]
    gxp = g_ref[0]
    gx = gxp[:, 0, :]
    gy = gxp[:, 1, :]
    gz = gxp[:, 2, :]
    cnt = cnt_ref[0]

    slot = lax.broadcasted_iota(jnp.int32, (BN, M), 1).astype(jnp.float32)
    mask = (slot < cnt - 0.5).astype(jnp.float32)

    rx = gx - qx
    ry = gy - qy
    rz = gz - qz
    d2m = (rx * rx + ry * ry) + rz * rz
    dist = jnp.sqrt(d2m + 1e-12)
    invd = 1.0 / (dist + 1e-8)
    cntm = jnp.maximum(jnp.sum(mask, axis=1, keepdims=True), 1.0)
    cx = jnp.sum(rx * mask, axis=1, keepdims=True) / cntm
    cy = jnp.sum(ry * mask, axis=1, keepdims=True) / cntm
    cz = jnp.sum(rz * mask, axis=1, keepdims=True) / cntm
    ones = jnp.ones_like(rx)
    feats = [
        rx / RADIUS,
        ry / RADIUS,
        rz / RADIUS,
        dist / RADIUS,
        rx * invd,
        ry * invd,
        rz * invd,
        ry * cz - rz * cy,
        rz * cx - rx * cz,
        rx * cy - ry * cx,
        (rx * cx + ry * cy) + rz * cz,
        rx * rx / R2,
        ry * ry / R2,
        rz * rz / R2,
        cx * ones,
        cy * ones,
        cz * ones,
        jnp.sqrt((cx * cx + cy * cy) + cz * cz + 1e-12) * ones,
        dist * dist / R2,
    ]
    zrow = jnp.zeros((BN, 1, M), jnp.float32)
    planes = []
    srows = []
    qrows = []
    for f in feats:
        fm = f * mask
        planes.append(fm[:, None, :])
        srows.append(jnp.sum(fm, axis=0, keepdims=True))
        qrows.append(jnp.sum(fm * fm, axis=0, keepdims=True))
    for _ in range(NGEO_PAD - NGEO):
        planes.append(zrow)
        srows.append(jnp.zeros((1, M), jnp.float32))
        qrows.append(jnp.zeros((1, M), jnp.float32))
    feats_ref[0] = jnp.concatenate(planes, axis=1)

    stats_upd = jnp.concatenate(srows + qrows, axis=0)
    first = (pl.program_id(0) == 0) & (pl.program_id(1) == 0)

    @pl.when(first)
    def _():
        stats_ref[...] = jnp.zeros_like(stats_ref)

    stats_ref[...] += stats_upd


def _pass_b2_body(g_ref, f_ref, wr_ref, wf_ref, wfi_ref, bias_ref,
                  y_ref, stats_ref):
    BN = g_ref.shape[1]
    CP = g_ref.shape[3]
    g = g_ref[0]
    ce = g[:, 0:1, :]
    r = (g - ce).reshape(BN * M, CP)
    f = f_ref[0].reshape(BN * M, NGEO_PAD)
    y = (jnp.dot(r, wr_ref[...], preferred_element_type=jnp.float32,
                 precision=lax.Precision.HIGHEST)
         + jnp.dot(f, wf_ref[...], preferred_element_type=jnp.float32,
                   precision=lax.Precision.HIGHEST))
    ceo = jnp.dot(g[:, 0, :], wfi_ref[...],
                  preferred_element_type=jnp.float32,
                  precision=lax.Precision.HIGHEST)
    y3 = (y.reshape(BN, M, 64) + ceo[:, None, :]
          + bias_ref[0:1, :][None, :, :])

    ymax = y3[:, 0, :]
    for m in range(1, M):
        ymax = jnp.maximum(ymax, y3[:, m, :])
    y_ref[0] = ymax

    y2 = y3.reshape(BN * M, 64)
    s = jnp.sum(y2, axis=0, keepdims=True)
    ss = jnp.sum(y2 * y2, axis=0, keepdims=True)
    upd = jnp.concatenate(
        [s, ss, jnp.zeros((6, 64), jnp.float32)], axis=0)
    first = (pl.program_id(0) == 0) & (pl.program_id(1) == 0)

    @pl.when(first)
    def _():
        stats_ref[...] = jnp.zeros_like(stats_ref)

    stats_ref[...] += upd


def _pass_c_body(y_ref, scsh_ref, o_ref):
    sc = scsh_ref[0:1, :]
    sh = scsh_ref[1:2, :]
    o_ref[0] = jnp.maximum(y_ref[0] * sc + sh, 0.0)


def _sc_gather(table, flat_idx):
    TOT = flat_idx.shape[0]
    D = table.shape[1]
    per_w = TOT // SC_WORKERS
    nchunks = per_w // SC_CHUNK
    mesh = plsc.VectorSubcoreMesh(core_axis_name="c", subcore_axis_name="s")

    @functools.partial(
        pl.kernel,
        out_type=jax.ShapeDtypeStruct((TOT, D), jnp.float32),
        mesh=mesh,
        scratch_types=[
            pltpu.VMEM((SC_CHUNK,), jnp.int32),
            pltpu.VMEM((SC_CHUNK, D), jnp.float32),
            pltpu.SemaphoreType.DMA,
        ],
    )
    def gather_k(table_hbm, idx_hbm, out_hbm, idx_v, rows_v, sem):
        wid = lax.axis_index("s") * SC_CORES + lax.axis_index("c")
        base = wid * per_w

        def body(k, carry):
            start = base + k * SC_CHUNK
            pltpu.sync_copy(idx_hbm.at[pl.ds(start, SC_CHUNK)], idx_v)
            pltpu.async_copy(table_hbm.at[idx_v], rows_v, sem).wait()
            pltpu.sync_copy(rows_v, out_hbm.at[pl.ds(start, SC_CHUNK)])
            return carry

        lax.fori_loop(0, nchunks, body, 0)

    return gather_k(table, flat_idx)


def kernel(q_xyz, sup_xyz, q_mask, sup_mask, sup_feats,
           W_fi1, W_fi2, W_fi3, W_df1, W_df2, W_df3):
    B, N, _ = q_xyz.shape
    NS = sup_xyz.shape[1]
    C = sup_feats.shape[1]
    f32 = jnp.float32

    supT = jnp.transpose(sup_xyz, (0, 2, 1))

    nb_a = N // BN_A
    idx, cnt = pl.pallas_call(
        _pass_a_body,
        grid=(B, nb_a),
        in_specs=[
            pl.BlockSpec((1, BN_A, 3), lambda b, n: (b, n, 0)),
            pl.BlockSpec((1, 3, NS), lambda b, n: (b, 0, 0)),
        ],
        out_specs=[
            pl.BlockSpec((1, BN_A, M), lambda b, n: (b, n, 0)),
            pl.BlockSpec((1, BN_A, 1), lambda b, n: (b, n, 0)),
        ],
        out_shape=[
            jax.ShapeDtypeStruct((B, N, M), jnp.int32),
            jax.ShapeDtypeStruct((B, N, 1), f32),
        ],
    )(q_xyz, supT)

    table = jnp.zeros((B * NS, 128), f32)
    table = table.at[:, 0:C].set(
        jnp.transpose(sup_feats, (0, 2, 1)).reshape(B * NS, C))
    table = table.at[:, C:C + 3].set(sup_xyz.reshape(B * NS, 3))
    g = _sc_gather(table, idx.reshape(B * N * M))
    gfeat = g.reshape(B, N, M, 128)
    gxyzT = jnp.transpose(gfeat[:, :, :, C:C + 3], (0, 1, 3, 2))

    nb_b = N // BN_B
    feats_cm, stats1 = pl.pallas_call(
        _pass_b1_body,
        grid=(B, nb_b),
        in_specs=[
            pl.BlockSpec((1, BN_B, 3), lambda b, n: (b, n, 0)),
            pl.BlockSpec((1, BN_B, 3, M), lambda b, n: (b, n, 0, 0)),
            pl.BlockSpec((1, BN_B, 1), lambda b, n: (b, n, 0)),
        ],
        out_specs=[
            pl.BlockSpec((1, BN_B, NGEO_PAD, M), lambda b, n: (b, n, 0, 0)),
            pl.BlockSpec((48, M), lambda b, n: (0, 0)),
        ],
        out_shape=[
            jax.ShapeDtypeStruct((B, N, NGEO_PAD, M), f32),
            jax.ShapeDtypeStruct((48, M), f32),
        ],
    )(q_xyz, gxyzT, cnt)

    cnt_tot = float(B * N * M)
    s1 = jnp.sum(stats1[0:NGEO_PAD], axis=1)[:NGEO]
    q1 = jnp.sum(stats1[NGEO_PAD:2 * NGEO_PAD], axis=1)[:NGEO]
    mu1 = s1 / cnt_tot
    var1 = q1 / cnt_tot - mu1 * mu1
    inv1 = 1.0 / jnp.sqrt(var1 + EPS_BN)

    c3o = W_fi1.shape[0]
    co3 = W_fi3.shape[0]
    c3i = W_fi1.shape[1]
    ci3 = W_fi3.shape[1]
    Cout = 2 * c3o + co3

    WrT = jnp.zeros((128, Cout), f32)
    WrT = WrT.at[0:c3i, 0:c3o].set(W_df1[:, 0:c3i].T)
    WrT = WrT.at[c3i:2 * c3i, c3o:2 * c3o].set(W_df2[:, 0:c3i].T)
    WrT = WrT.at[2 * c3i:C, 2 * c3o:Cout].set(W_df3[:, 0:ci3].T)

    WfT = jnp.zeros((NGEO_PAD, Cout), f32)
    WfT = WfT.at[0:3, 0:c3o].set((W_df1[:, c3i:c3i + 3] * inv1[None, 0:3]).T)
    WfT = WfT.at[3:7, c3o:2 * c3o].set(
        (W_df2[:, c3i:c3i + 4] * inv1[None, 3:7]).T)
    WfT = WfT.at[7:NGEO, 2 * c3o:Cout].set(
        (W_df3[:, ci3:ci3 + 12] * inv1[None, 7:NGEO]).T)
    mu_pad = jnp.zeros((NGEO_PAD,), f32).at[0:NGEO].set(mu1)
    bias = -(mu_pad @ WfT)
    bias8 = jnp.zeros((8, Cout), f32).at[0, :].set(bias)

    WfiT = jnp.zeros((128, Cout), f32)
    WfiT = WfiT.at[0:c3i, 0:c3o].set(W_fi1.T)
    WfiT = WfiT.at[c3i:2 * c3i, c3o:2 * c3o].set(W_fi2.T)
    WfiT = WfiT.at[2 * c3i:C, 2 * c3o:Cout].set(W_fi3.T)

    feats24 = jnp.transpose(feats_cm, (0, 1, 3, 2))

    y, stats2 = pl.pallas_call(
        _pass_b2_body,
        grid=(B, nb_b),
        in_specs=[
            pl.BlockSpec((1, BN_B, M, 128), lambda b, n: (b, n, 0, 0)),
            pl.BlockSpec((1, BN_B, M, NGEO_PAD), lambda b, n: (b, n, 0, 0)),
            pl.BlockSpec((128, Cout), lambda b, n: (0, 0)),
            pl.BlockSpec((NGEO_PAD, Cout), lambda b, n: (0, 0)),
            pl.BlockSpec((128, Cout), lambda b, n: (0, 0)),
            pl.BlockSpec((8, Cout), lambda b, n: (0, 0)),
        ],
        out_specs=[
            pl.BlockSpec((1, BN_B, Cout), lambda b, n: (b, n, 0)),
            pl.BlockSpec((8, Cout), lambda b, n: (0, 0)),
        ],
        out_shape=[
            jax.ShapeDtypeStruct((B, N, Cout), f32),
            jax.ShapeDtypeStruct((8, Cout), f32),
        ],
    )(gfeat, feats24, WrT, WfT, WfiT, bias8)

    mu2 = stats2[0] / cnt_tot
    var2 = stats2[1] / cnt_tot - mu2 * mu2
    sc2 = 1.0 / jnp.sqrt(var2 + EPS_BN)
    scsh = jnp.zeros((8, Cout), f32).at[0, :].set(sc2).at[1, :].set(-mu2 * sc2)

    z = pl.pallas_call(
        _pass_c_body,
        grid=(B,),
        in_specs=[
            pl.BlockSpec((1, N, Cout), lambda b: (b, 0, 0)),
            pl.BlockSpec((8, Cout), lambda b: (0, 0)),
        ],
        out_specs=pl.BlockSpec((1, N, Cout), lambda b: (b, 0, 0)),
        out_shape=jax.ShapeDtypeStruct((B, N, Cout), f32),
    )(y, scsh)

    return jnp.transpose(z, (0, 2, 1))


# speedup vs baseline: 13.9236x; 1.6784x over previous
import functools

import jax
import jax.numpy as jnp
from jax import lax
from jax.experimental import pallas as pl
from jax.experimental.pallas import tpu as pltpu
from jax.experimental.pallas import tpu_sc as plsc

RADIUS = 0.2
R2 = RADIUS * RADIUS
M = 32
NGEO = 19
NGEO_PAD = 24
EPS_BN = 1e-5

BN_A = 256
BN_B = 128
CHUNK = 128

SC_CORES = 2
SC_SUBCORES = 16
SC_WORKERS = SC_CORES * SC_SUBCORES
SC_CHUNK = 256


def _pass_a_body(q_ref, supT_ref, idx_ref, cnt_ref):
    b = pl.program_id(0)
    BN = q_ref.shape[1]
    NS = supT_ref.shape[2]

    q = q_ref[0]
    qx = q[:, 0:1]
    qy = q[:, 1:2]
    qz = q[:, 2:3]
    sx = supT_ref[0, 0:1, :]
    sy = supT_ref[0, 1:2, :]
    sz = supT_ref[0, 2:3, :]

    dx = qx - sx
    dy = qy - sy
    dz = qz - sz
    d2 = (dx * dx + dy * dy) + dz * dz
    valid = (d2 <= R2).astype(jnp.bfloat16)

    nchunks = NS // CHUNK
    row_i = lax.broadcasted_iota(jnp.int32, (CHUNK, CHUNK), 0)
    col_i = lax.broadcasted_iota(jnp.int32, (CHUNK, CHUNK), 1)
    lt = (row_i <= col_i).astype(jnp.bfloat16)
    off = jnp.zeros((BN, 1), jnp.float32)
    rank_chunks = []
    for c in range(nchunks):
        vc = valid[:, c * CHUNK:(c + 1) * CHUNK]
        rc = jnp.dot(vc, lt, preferred_element_type=jnp.float32) + off
        rank_chunks.append(rc)
        off = rc[:, CHUNK - 1:CHUNK]
    rank = jnp.concatenate(rank_chunks, axis=1)
    cnt = off

    rcl = jnp.minimum(rank, 33.0)
    idx_cols = []
    for m in range(M):
        cmp = (rcl <= (m + 0.5)).astype(jnp.float32)
        idx_cols.append(jnp.sum(cmp, axis=1, keepdims=True))
    idxm = jnp.concatenate(idx_cols, axis=1)

    slot = lax.broadcasted_iota(jnp.int32, (BN, M), 1).astype(jnp.float32)
    mask = (slot < cnt - 0.5).astype(jnp.float32)

    idx0 = jnp.where(cnt > 0.5, idxm[:, 0:1], 0.0)
    idx_fix = jnp.where(mask > 0.5, idxm, idx0)
    flat = idx_fix + b.astype(jnp.float32) * float(NS) + 0.5
    idx_ref[0] = flat.astype(jnp.int32)
    cnt_ref[0] = cnt


def _pass_b1_body(qT_ref, g_ref, cnt_ref, feats_ref, stats_ref):
    BN = qT_ref.shape[2]
    qx = qT_ref[0, 0:1, :]
    qy = qT_ref[0, 1:2, :]
    qz = qT_ref[0, 2:3, :]
    gxp = g_ref[0]
    gx = gxp[0]
    gy = gxp[1]
    gz = gxp[2]
    cnt = cnt_ref[0]

    slot = lax.broadcasted_iota(jnp.int32, (M, BN), 0).astype(jnp.float32)
    mask = (slot < cnt - 0.5).astype(jnp.float32)

    rx = gx - qx
    ry = gy - qy
    rz = gz - qz
    d2m = (rx * rx + ry * ry) + rz * rz
    dist = jnp.sqrt(d2m + 1e-12)
    invd = 1.0 / (dist + 1e-8)
    cntm = jnp.maximum(jnp.sum(mask, axis=0, keepdims=True), 1.0)
    cx = jnp.sum(rx * mask, axis=0, keepdims=True) / cntm
    cy = jnp.sum(ry * mask, axis=0, keepdims=True) / cntm
    cz = jnp.sum(rz * mask, axis=0, keepdims=True) / cntm
    ones = jnp.ones_like(rx)
    feats = [
        rx / RADIUS,
        ry / RADIUS,
        rz / RADIUS,
        dist / RADIUS,
        rx * invd,
        ry * invd,
        rz * invd,
        ry * cz - rz * cy,
        rz * cx - rx * cz,
        rx * cy - ry * cx,
        (rx * cx + ry * cy) + rz * cz,
        rx * rx / R2,
        ry * ry / R2,
        rz * rz / R2,
        cx * ones,
        cy * ones,
        cz * ones,
        jnp.sqrt((cx * cx + cy * cy) + cz * cz + 1e-12) * ones,
        dist * dist / R2,
    ]
    zplane = jnp.zeros((1, M, BN), jnp.float32)
    planes = []
    srows = []
    qrows = []
    for f in feats:
        fm = f * mask
        planes.append(fm[None, :, :])
        srows.append(jnp.sum(fm, axis=0, keepdims=True))
        qrows.append(jnp.sum(fm * fm, axis=0, keepdims=True))
    for _ in range(NGEO_PAD - NGEO):
        planes.append(zplane)
        srows.append(jnp.zeros((1, BN), jnp.float32))
        qrows.append(jnp.zeros((1, BN), jnp.float32))
    feats_ref[0] = jnp.concatenate(planes, axis=0)

    stats_upd = jnp.concatenate(srows + qrows, axis=0)
    first = (pl.program_id(0) == 0) & (pl.program_id(1) == 0)

    @pl.when(first)
    def _():
        stats_ref[...] = jnp.zeros_like(stats_ref)

    stats_ref[...] += stats_upd


def _pass_b2_body(g_ref, f_ref, wr_ref, wf_ref, wfi_ref, bias_ref,
                  y_ref, stats_ref):
    BN = g_ref.shape[1]
    CP = g_ref.shape[3]
    g = g_ref[0]
    ce = g[:, 0:1, :]
    r = (g - ce).reshape(BN * M, CP)
    f = f_ref[0].reshape(BN * M, NGEO_PAD)
    y = (jnp.dot(r, wr_ref[...], preferred_element_type=jnp.float32)
         + jnp.dot(f, wf_ref[...], preferred_element_type=jnp.float32))
    ceo = jnp.dot(g[:, 0, :], wfi_ref[...],
                  preferred_element_type=jnp.float32)
    y3 = (y.reshape(BN, M, 64) + ceo[:, None, :]
          + bias_ref[0:1, :][None, :, :])

    ym = y3
    w = M
    while w > 1:
        half = w // 2
        ym = jnp.maximum(ym[:, :half, :], ym[:, half:w, :])
        w = half
    y_ref[0] = ym[:, 0, :]

    y2 = y3.reshape(BN * M, 64)
    s = jnp.sum(y2, axis=0, keepdims=True)
    ss = jnp.sum(y2 * y2, axis=0, keepdims=True)
    upd = jnp.concatenate(
        [s, ss, jnp.zeros((6, 64), jnp.float32)], axis=0)
    first = (pl.program_id(0) == 0) & (pl.program_id(1) == 0)

    @pl.when(first)
    def _():
        stats_ref[...] = jnp.zeros_like(stats_ref)

    stats_ref[...] += upd


def _pass_c_body(y_ref, scsh_ref, o_ref):
    sc = scsh_ref[0:1, :]
    sh = scsh_ref[1:2, :]
    o_ref[0] = jnp.maximum(y_ref[0] * sc + sh, 0.0)


def _sc_gather(table, flat_idx):
    TOT = flat_idx.shape[0]
    D = table.shape[1]
    C = 64
    per_w = TOT // SC_WORKERS
    nchunks = per_w // SC_CHUNK
    mesh = plsc.VectorSubcoreMesh(core_axis_name="c", subcore_axis_name="s")

    @functools.partial(
        pl.kernel,
        out_type=jax.ShapeDtypeStruct((TOT, D), jnp.float32),
        mesh=mesh,
        scratch_types=[
            pltpu.VMEM((SC_CHUNK,), jnp.int32),
            pltpu.VMEM((SC_CHUNK,), jnp.int32),
            pltpu.VMEM((SC_CHUNK, D), jnp.float32),
            pltpu.VMEM((SC_CHUNK, D), jnp.float32),
            pltpu.SemaphoreType.DMA,
            pltpu.SemaphoreType.DMA,
        ],
    )
    def gather_k(table_hbm, idx_hbm, out_hbm,
                 idx_v0, idx_v1, rows_v0, rows_v1, sem0, sem1):
        wid = lax.axis_index("s") * SC_CORES + lax.axis_index("c")
        base = wid * per_w

        def body(t, carry):
            s0 = base + (2 * t) * SC_CHUNK
            s1 = s0 + SC_CHUNK
            pltpu.sync_copy(idx_hbm.at[pl.ds(s0, SC_CHUNK)], idx_v0)
            cp0 = pltpu.async_copy(table_hbm.at[idx_v0], rows_v0, sem0)
            pltpu.sync_copy(idx_hbm.at[pl.ds(s1, SC_CHUNK)], idx_v1)
            cp1 = pltpu.async_copy(table_hbm.at[idx_v1], rows_v1, sem1)
            cp0.wait()
            pltpu.sync_copy(rows_v0, out_hbm.at[pl.ds(s0, SC_CHUNK)])
            cp1.wait()
            pltpu.sync_copy(rows_v1, out_hbm.at[pl.ds(s1, SC_CHUNK)])
            return carry

        lax.fori_loop(0, nchunks // 2, body, 0)

    return gather_k(table, flat_idx)


def kernel(q_xyz, sup_xyz, q_mask, sup_mask, sup_feats,
           W_fi1, W_fi2, W_fi3, W_df1, W_df2, W_df3):
    B, N, _ = q_xyz.shape
    NS = sup_xyz.shape[1]
    C = sup_feats.shape[1]
    f32 = jnp.float32

    supT = jnp.transpose(sup_xyz, (0, 2, 1))

    nb_a = N // BN_A
    idx, cnt = pl.pallas_call(
        _pass_a_body,
        grid=(B, nb_a),
        in_specs=[
            pl.BlockSpec((1, BN_A, 3), lambda b, n: (b, n, 0)),
            pl.BlockSpec((1, 3, NS), lambda b, n: (b, 0, 0)),
        ],
        out_specs=[
            pl.BlockSpec((1, BN_A, M), lambda b, n: (b, n, 0)),
            pl.BlockSpec((1, BN_A, 1), lambda b, n: (b, n, 0)),
        ],
        out_shape=[
            jax.ShapeDtypeStruct((B, N, M), jnp.int32),
            jax.ShapeDtypeStruct((B, N, 1), f32),
        ],
    )(q_xyz, supT)

    table = jnp.zeros((B * NS, 128), f32)
    table = table.at[:, 0:C].set(
        jnp.transpose(sup_feats, (0, 2, 1)).reshape(B * NS, C))
    table = table.at[:, C:C + 3].set(sup_xyz.reshape(B * NS, 3))
    g = _sc_gather(table, idx.reshape(B * N * M))
    gfeat = g.reshape(B, N, M, 128)
    gxyzT = jnp.transpose(gfeat[:, :, :, C:C + 3], (0, 3, 2, 1))
    qT = jnp.transpose(q_xyz, (0, 2, 1))
    cntT = cnt.reshape(B, 1, N)

    nb_b = N // BN_B
    feats_cm, stats1 = pl.pallas_call(
        _pass_b1_body,
        grid=(B, nb_b),
        in_specs=[
            pl.BlockSpec((1, 3, BN_B), lambda b, n: (b, 0, n)),
            pl.BlockSpec((1, 3, M, BN_B), lambda b, n: (b, 0, 0, n)),
            pl.BlockSpec((1, 1, BN_B), lambda b, n: (b, 0, n)),
        ],
        out_specs=[
            pl.BlockSpec((1, NGEO_PAD, M, BN_B), lambda b, n: (b, 0, 0, n)),
            pl.BlockSpec((48, BN_B), lambda b, n: (0, 0)),
        ],
        out_shape=[
            jax.ShapeDtypeStruct((B, NGEO_PAD, M, N), f32),
            jax.ShapeDtypeStruct((48, BN_B), f32),
        ],
    )(qT, gxyzT, cntT)

    cnt_tot = float(B * N * M)
    s1 = jnp.sum(stats1[0:NGEO_PAD], axis=1)[:NGEO]
    q1 = jnp.sum(stats1[NGEO_PAD:2 * NGEO_PAD], axis=1)[:NGEO]
    mu1 = s1 / cnt_tot
    var1 = q1 / cnt_tot - mu1 * mu1
    inv1 = 1.0 / jnp.sqrt(var1 + EPS_BN)

    c3o = W_fi1.shape[0]
    co3 = W_fi3.shape[0]
    c3i = W_fi1.shape[1]
    ci3 = W_fi3.shape[1]
    Cout = 2 * c3o + co3

    WrT = jnp.zeros((128, Cout), f32)
    WrT = WrT.at[0:c3i, 0:c3o].set(W_df1[:, 0:c3i].T)
    WrT = WrT.at[c3i:2 * c3i, c3o:2 * c3o].set(W_df2[:, 0:c3i].T)
    WrT = WrT.at[2 * c3i:C, 2 * c3o:Cout].set(W_df3[:, 0:ci3].T)

    WfT = jnp.zeros((NGEO_PAD, Cout), f32)
    WfT = WfT.at[0:3, 0:c3o].set((W_df1[:, c3i:c3i + 3] * inv1[None, 0:3]).T)
    WfT = WfT.at[3:7, c3o:2 * c3o].set(
        (W_df2[:, c3i:c3i + 4] * inv1[None, 3:7]).T)
    WfT = WfT.at[7:NGEO, 2 * c3o:Cout].set(
        (W_df3[:, ci3:ci3 + 12] * inv1[None, 7:NGEO]).T)
    mu_pad = jnp.zeros((NGEO_PAD,), f32).at[0:NGEO].set(mu1)
    bias = -(mu_pad @ WfT)
    bias8 = jnp.zeros((8, Cout), f32).at[0, :].set(bias)

    WfiT = jnp.zeros((128, Cout), f32)
    WfiT = WfiT.at[0:c3i, 0:c3o].set(W_fi1.T)
    WfiT = WfiT.at[c3i:2 * c3i, c3o:2 * c3o].set(W_fi2.T)
    WfiT = WfiT.at[2 * c3i:C, 2 * c3o:Cout].set(W_fi3.T)

    feats24 = jnp.transpose(feats_cm, (0, 3, 2, 1))

    y, stats2 = pl.pallas_call(
        _pass_b2_body,
        grid=(B, nb_b),
        in_specs=[
            pl.BlockSpec((1, BN_B, M, 128), lambda b, n: (b, n, 0, 0)),
            pl.BlockSpec((1, BN_B, M, NGEO_PAD), lambda b, n: (b, n, 0, 0)),
            pl.BlockSpec((128, Cout), lambda b, n: (0, 0)),
            pl.BlockSpec((NGEO_PAD, Cout), lambda b, n: (0, 0)),
            pl.BlockSpec((128, Cout), lambda b, n: (0, 0)),
            pl.BlockSpec((8, Cout), lambda b, n: (0, 0)),
        ],
        out_specs=[
            pl.BlockSpec((1, BN_B, Cout), lambda b, n: (b, n, 0)),
            pl.BlockSpec((8, Cout), lambda b, n: (0, 0)),
        ],
        out_shape=[
            jax.ShapeDtypeStruct((B, N, Cout), f32),
            jax.ShapeDtypeStruct((8, Cout), f32),
        ],
    )(gfeat, feats24, WrT, WfT, WfiT, bias8)

    mu2 = stats2[0] / cnt_tot
    var2 = stats2[1] / cnt_tot - mu2 * mu2
    sc2 = 1.0 / jnp.sqrt(var2 + EPS_BN)
    scsh = jnp.zeros((8, Cout), f32).at[0, :].set(sc2).at[1, :].set(-mu2 * sc2)

    z = pl.pallas_call(
        _pass_c_body,
        grid=(B,),
        in_specs=[
            pl.BlockSpec((1, N, Cout), lambda b: (b, 0, 0)),
            pl.BlockSpec((8, Cout), lambda b: (0, 0)),
        ],
        out_specs=pl.BlockSpec((1, N, Cout), lambda b: (b, 0, 0)),
        out_shape=jax.ShapeDtypeStruct((B, N, Cout), f32),
    )(y, scsh)

    return jnp.transpose(z, (0, 2, 1))
